# Initial kernel scaffold; baseline (speedup 1.0000x reference)
import jax
import jax.numpy as jnp
from jax.experimental import pallas as pl


def kernel(inputs, token_table, pos_table):
    raise NotImplementedError("write your pallas kernel here")


# speedup vs baseline: 1.3894x; 1.3894x over previous
import functools

import jax
import jax.numpy as jnp
from jax import lax
from jax.experimental import pallas as pl
from jax.experimental.pallas import tpu as pltpu
from jax.experimental.pallas import tpu_sc as plsc

D = 32
SEQ = 200
G = 100
CHUNK_SEQS = 4
CHUNK = CHUNK_SEQS * SEQ
GPC = CHUNK // G


def kernel(inputs, token_table, pos_table):
    B, S = inputs.shape
    V, d = token_table.shape
    assert S == SEQ and d == D
    n_rows = B * S

    info = plsc.get_sparse_core_info()
    NC, NS = info.num_cores, info.num_subcores
    NW = NC * NS
    assert n_rows % (NW * CHUNK) == 0
    chunks_per_worker = n_rows // (NW * CHUNK)

    idx2d = inputs.reshape(n_rows // G, G).astype(jnp.int32)

    mesh = plsc.VectorSubcoreMesh(core_axis_name="c", subcore_axis_name="s")

    @functools.partial(
        pl.kernel,
        mesh=mesh,
        compiler_params=pltpu.CompilerParams(use_tc_tiling_on_sc=False),
        out_type=jax.ShapeDtypeStruct((n_rows, D), jnp.float32),
        scratch_types=[
            pltpu.VMEM((GPC, G), jnp.int32),
            pltpu.VMEM((CHUNK, D), jnp.float32),
            pltpu.VMEM((SEQ, D), jnp.float32),
            pltpu.SemaphoreType.DMA,
        ],
    )
    def body(idx_hbm, tok_hbm, pos_hbm, out_hbm, idx_v, buf_v, pos_v, sem):
        wid = lax.axis_index("s") * NC + lax.axis_index("c")
        pltpu.sync_copy(pos_hbm, pos_v)

        def do_chunk(c, _):
            chunk_id = wid * chunks_per_worker + c
            row0 = chunk_id * CHUNK
            grow0 = chunk_id * GPC
            pltpu.sync_copy(idx_hbm.at[pl.ds(grow0, GPC)], idx_v)
            copies = []
            for j in range(GPC):
                copies.append(
                    pltpu.async_copy(
                        tok_hbm.at[idx_v.at[j]],
                        buf_v.at[pl.ds(j * G, G)],
                        sem,
                    )
                )
            for cp in copies:
                cp.wait()

            def add_pos(t, _):
                p0 = pos_v[t, pl.ds(0, 16)]
                p1 = pos_v[t, pl.ds(16, 16)]
                for s_i in range(CHUNK_SEQS):
                    r = s_i * SEQ + t
                    buf_v[r, pl.ds(0, 16)] = buf_v[r, pl.ds(0, 16)] + p0
                    buf_v[r, pl.ds(16, 16)] = buf_v[r, pl.ds(16, 16)] + p1
                return 0

            lax.fori_loop(0, SEQ, add_pos, 0)
            pltpu.sync_copy(buf_v, out_hbm.at[pl.ds(row0, CHUNK)])
            return 0

        lax.fori_loop(0, chunks_per_worker, do_chunk, 0)

    out = body(idx2d, token_table, pos_table)
    return out.reshape(B, S, D)


# speedup vs baseline: 1.4876x; 1.0707x over previous
import functools

import jax
import jax.numpy as jnp
from jax import lax
from jax.experimental import pallas as pl
from jax.experimental.pallas import tpu as pltpu
from jax.experimental.pallas import tpu_sc as plsc

D = 32
SEQ = 200
G = 100
CHUNK_SEQS = 4
CHUNK = CHUNK_SEQS * SEQ
GPC = CHUNK // G


def kernel(inputs, token_table, pos_table):
    B, S = inputs.shape
    V, d = token_table.shape
    assert S == SEQ and d == D
    n_rows = B * S

    info = plsc.get_sparse_core_info()
    NC, NS = info.num_cores, info.num_subcores
    NW = NC * NS
    assert n_rows % (NW * CHUNK) == 0
    cpw = n_rows // (NW * CHUNK)
    assert cpw % 2 == 0
    rows_pw = cpw * GPC

    idx2d = inputs.reshape(n_rows // G, G).astype(jnp.int32)

    mesh = plsc.VectorSubcoreMesh(core_axis_name="c", subcore_axis_name="s")

    @functools.partial(
        pl.kernel,
        mesh=mesh,
        compiler_params=pltpu.CompilerParams(use_tc_tiling_on_sc=False),
        out_type=jax.ShapeDtypeStruct((n_rows, D), jnp.float32),
        scratch_types=[
            pltpu.VMEM((rows_pw, G), jnp.int32),
            pltpu.VMEM((CHUNK, D), jnp.float32),
            pltpu.VMEM((CHUNK, D), jnp.float32),
            pltpu.VMEM((SEQ, D), jnp.float32),
            pltpu.SemaphoreType.DMA,
            pltpu.SemaphoreType.DMA,
        ],
    )
    def body(idx_hbm, tok_hbm, pos_hbm, out_hbm,
             idx_all, buf0, buf1, pos_v, sem0, sem1):
        wid = lax.axis_index("s") * NC + lax.axis_index("c")
        pltpu.sync_copy(pos_hbm, pos_v)
        pltpu.sync_copy(idx_hbm.at[pl.ds(wid * rows_pw, rows_pw)], idx_all)

        def fire(c, buf, sem):
            for j in range(GPC):
                pltpu.async_copy(
                    tok_hbm.at[idx_all.at[c * GPC + j]],
                    buf.at[pl.ds(j * G, G)],
                    sem,
                )

        def drain(buf, sem):
            pltpu.make_async_copy(
                tok_hbm.at[pl.ds(0, CHUNK)], buf, sem
            ).wait()

        def add_pos(buf):
            def step(t, _):
                p0 = pos_v[t, pl.ds(0, 16)]
                p1 = pos_v[t, pl.ds(16, 16)]
                for s_i in range(CHUNK_SEQS):
                    r = s_i * SEQ + t
                    buf[r, pl.ds(0, 16)] = buf[r, pl.ds(0, 16)] + p0
                    buf[r, pl.ds(16, 16)] = buf[r, pl.ds(16, 16)] + p1
                return 0

            lax.fori_loop(0, SEQ, step, 0)

        def writeback(c, buf):
            row0 = (wid * cpw + c) * CHUNK
            pltpu.sync_copy(buf, out_hbm.at[pl.ds(row0, CHUNK)])

        fire(0, buf0, sem0)

        def pair(p, _):
            c0 = 2 * p
            c1 = c0 + 1
            fire(c1, buf1, sem1)
            drain(buf0, sem0)
            add_pos(buf0)
            writeback(c0, buf0)

            @pl.when(c1 + 1 < cpw)
            def _():
                fire(c1 + 1, buf0, sem0)

            drain(buf1, sem1)
            add_pos(buf1)
            writeback(c1, buf1)
            return 0

        lax.fori_loop(0, cpw // 2, pair, 0)

    out = body(idx2d, token_table, pos_table)
    return out.reshape(B, S, D)
